# baseline (speedup 1.0000x reference)
import functools

import jax
import jax.numpy as jnp
from jax import lax
from jax.experimental import pallas as pl
from jax.experimental.pallas import tpu as pltpu
from jax.experimental.pallas import tpu_sc as plsc

N_NODES = 10000
N_EDGES = 320000
F = 128

NC = 2
NS = 16
L = 16

EDGES_PER_TILE = N_EDGES // (NC * NS)
SUB = 80
NSUB = 2
CHUNK = NSUB * SUB
N_FULL = EDGES_PER_TILE // CHUNK
TAIL_SUBS = (EDGES_PER_TILE - N_FULL * CHUNK) // SUB
ROWS_PER_TILE = 624
TAIL_ROWS = N_NODES - NS * ROWS_PER_TILE


def _mm_body(x_ref, w_ref, o_ref):
    o_ref[...] = jnp.dot(x_ref[...], w_ref[...], preferred_element_type=jnp.float32)


def _matmul(x, W):
    return pl.pallas_call(
        _mm_body,
        grid=(10,),
        in_specs=[
            pl.BlockSpec((1000, F), lambda i: (i, 0)),
            pl.BlockSpec((F, F), lambda i: (0, 0)),
        ],
        out_specs=pl.BlockSpec((1000, F), lambda i: (i, 0)),
        out_shape=jax.ShapeDtypeStruct((N_NODES, F), jnp.float32),
    )(x, W)


_mesh = plsc.VectorSubcoreMesh(core_axis_name="c", subcore_axis_name="s")


@functools.partial(
    pl.kernel,
    out_type=jax.ShapeDtypeStruct((NC, N_NODES, F), jnp.float32),
    mesh=_mesh,
    scratch_types=[
        pltpu.VMEM((2, NSUB, SUB), jnp.int32),
        pltpu.VMEM((2, NSUB, SUB), jnp.int32),
        pltpu.VMEM((2, NSUB, SUB), jnp.float32),
        pltpu.VMEM((2, CHUNK, F), jnp.float32),
        pltpu.VMEM_SHARED((N_NODES, F), jnp.float32),
        pltpu.SemaphoreType.DMA,
        pltpu.SemaphoreType.DMA,
    ],
)
def _spmm(sup, col, row, w, out, col2, row2, w2, gb2, acc, gsem, isem):
    c = lax.axis_index("c")
    s = lax.axis_index("s")
    gid = c * NS + s
    tile_base = gid * EDGES_PER_TILE

    def idx_copies(i, slot, n):
        ds = []
        for sub in range(n):
            base = tile_base + i * CHUNK + sub * SUB
            ds += [
                pltpu.make_async_copy(col.at[pl.ds(base, SUB)], col2.at[slot, sub], isem),
                pltpu.make_async_copy(row.at[pl.ds(base, SUB)], row2.at[slot, sub], isem),
                pltpu.make_async_copy(w.at[pl.ds(base, SUB)], w2.at[slot, sub], isem),
            ]
        return ds

    def idx_load(i, slot, n):
        for d in idx_copies(i, slot, n):
            d.start()

    def idx_wait(i, slot, n):
        for d in idx_copies(i, slot, n):
            d.wait()

    def gather_copies(slot, n):
        return [
            pltpu.make_async_copy(sup.at[col2.at[slot, sub]],
                                  gb2.at[slot, pl.ds(sub * SUB, SUB)], gsem)
            for sub in range(n)
        ]

    def gather_start(slot, n):
        for d in gather_copies(slot, n):
            d.start()

    def gather_wait(slot, n):
        for d in gather_copies(slot, n):
            d.wait()

    def scale(slot, n):
        for sub in range(n):
            def body(j16, c2, sub=sub):
                wv = w2[slot, sub, pl.ds(j16 * L, L)]
                for k in range(L):
                    wj = jnp.broadcast_to(wv[k], (L,))
                    e = sub * SUB + j16 * L + k
                    for f in range(F // L):
                        sl = pl.ds(f * L, L)
                        gb2[slot, e, sl] = gb2[slot, e, sl] * wj
                return c2

            lax.fori_loop(0, SUB // L, body, 0)

    def scatter_add(slot, n):
        for sub in range(n):
            pltpu.sync_copy(gb2.at[slot, pl.ds(sub * SUB, SUB)],
                            acc.at[row2.at[slot, sub]], add=True)

    def step(i, slot, next_n, load_n):
        if next_n:
            idx_wait(i + 1, 1 - slot, next_n)
        gather_wait(slot, NSUB)
        if next_n:
            gather_start(1 - slot, next_n)
        scale(slot, NSUB)
        scatter_add(slot, NSUB)
        if load_n:
            idx_load(i + 2, slot, load_n)

    zero = jnp.zeros((L,), jnp.float32)

    def zrow(i, carry):
        for f in range(F // L):
            gb2[0, i, pl.ds(f * L, L)] = zero
        return carry

    lax.fori_loop(0, CHUNK, zrow, 0)
    r0 = s * ROWS_PER_TILE
    for off in range(0, ROWS_PER_TILE, CHUNK):
        sz = min(CHUNK, ROWS_PER_TILE - off)
        pltpu.sync_copy(gb2.at[0, pl.ds(0, sz)], acc.at[pl.ds(r0 + off, sz)])

    @pl.when(s == NS - 1)
    def _zero_tail():
        pltpu.sync_copy(gb2.at[0, pl.ds(0, TAIL_ROWS)],
                        acc.at[pl.ds(NS * ROWS_PER_TILE, TAIL_ROWS)])

    plsc.subcore_barrier()

    idx_load(0, 0, NSUB)
    idx_wait(0, 0, NSUB)
    gather_start(0, NSUB)
    idx_load(1, 1, NSUB)

    n_pairs = (N_FULL - 2) // 2

    def pair(p, carry):
        i0 = 2 * p
        step(i0, 0, NSUB, NSUB)
        step(i0 + 1, 1, NSUB, NSUB)
        return carry

    lax.fori_loop(0, n_pairs, pair, 0)

    i = 2 * n_pairs
    step(i, 0, NSUB, TAIL_SUBS)
    step(i + 1, 1, TAIL_SUBS, 0)
    gather_wait(0, TAIL_SUBS)
    scale(0, TAIL_SUBS)
    scatter_add(0, TAIL_SUBS)

    plsc.subcore_barrier()

    for off in range(0, ROWS_PER_TILE, CHUNK):
        sz = min(CHUNK, ROWS_PER_TILE - off)
        pltpu.sync_copy(acc.at[pl.ds(r0 + off, sz)], gb2.at[0, pl.ds(0, sz)])
        pltpu.sync_copy(gb2.at[0, pl.ds(0, sz)], out.at[c, pl.ds(r0 + off, sz)])

    @pl.when(s == NS - 1)
    def _write_tail():
        t0 = NS * ROWS_PER_TILE
        pltpu.sync_copy(acc.at[pl.ds(t0, TAIL_ROWS)], gb2.at[0, pl.ds(0, TAIL_ROWS)])
        pltpu.sync_copy(gb2.at[0, pl.ds(0, TAIL_ROWS)], out.at[c, pl.ds(t0, TAIL_ROWS)])


def _comb_body(p_ref, b_ref, o_ref):
    o_ref[...] = p_ref[0] + p_ref[1] + b_ref[...]


def _combine(partials, b2):
    return pl.pallas_call(
        _comb_body,
        grid=(10,),
        in_specs=[
            pl.BlockSpec((NC, 1000, F), lambda i: (0, i, 0)),
            pl.BlockSpec((1, F), lambda i: (0, 0)),
        ],
        out_specs=pl.BlockSpec((1000, F), lambda i: (i, 0)),
        out_shape=jax.ShapeDtypeStruct((N_NODES, F), jnp.float32),
    )(partials, b2)


def kernel(input, edge_index, edge_weight, W, b):
    ei = edge_index.astype(jnp.int32)
    row = ei[0]
    col = ei[1]
    sup = _matmul(input, W)
    partials = _spmm(sup, col, row, edge_weight)
    return _combine(partials, b.reshape(1, F))
